# baseline (speedup 1.0000x reference)
import functools
import math

import jax
import jax.numpy as jnp
from jax.experimental import pallas as pl
from jax.experimental.pallas import tpu as pltpu
from jax.experimental.pallas import tpu_sc as plsc

DIM = 1024
SCALE = math.sqrt(DIM)
WINDOW = 32
LANES = 16


def kernel(x, table):
    batch, seq = x.shape
    n = batch * seq
    idx = x.reshape(n).astype(jnp.int32)
    mesh = plsc.VectorSubcoreMesh(
        core_axis_name="core", subcore_axis_name="subcore"
    )

    @functools.partial(
        pl.kernel,
        out_type=jax.ShapeDtypeStruct((n, DIM), jnp.float32),
        mesh=mesh,
    )
    def emb_kernel(table_hbm, idx_hbm, out_hbm):
        def body(i_vmem, o_vmem):
            pltpu.sync_copy(table_hbm.at[i_vmem], o_vmem)

            @pl.loop(0, WINDOW)
            def _(r):
                @pl.loop(0, DIM, step=4 * LANES)
                def _(c):
                    for k in range(4):
                        sl = pl.ds(c + k * LANES, LANES)
                        o_vmem.at[r, sl][...] = o_vmem.at[r, sl][...] * SCALE

        pltpu.emit_pipeline(
            body,
            grid=(n // WINDOW,),
            in_specs=[pl.BlockSpec((WINDOW,), index_map=lambda i: (i,))],
            out_specs=[pl.BlockSpec((WINDOW, DIM), index_map=lambda i: (i, 0))],
            core_axis_name=("core", "subcore"),
            dimension_semantics=(pltpu.PARALLEL,),
        )(idx_hbm, out_hbm)

    out = emb_kernel(table, idx)
    return out.reshape(batch, seq, DIM)


# speedup vs baseline: 2.2709x; 2.2709x over previous
import functools
import math

import jax
import jax.numpy as jnp
from jax.experimental import pallas as pl
from jax.experimental.pallas import tpu as pltpu
from jax.experimental.pallas import tpu_sc as plsc

DIM = 1024
SCALE = math.sqrt(DIM)
WINDOW = 32
LANES = 16


def kernel(x, table):
    batch, seq = x.shape
    n = batch * seq
    idx = x.reshape(n).astype(jnp.int32)
    mesh = plsc.VectorSubcoreMesh(
        core_axis_name="core", subcore_axis_name="subcore"
    )

    @functools.partial(
        pl.kernel,
        out_type=jax.ShapeDtypeStruct((n, DIM), jnp.float32),
        mesh=mesh,
    )
    def emb_kernel(table_hbm, idx_hbm, out_hbm):
        def body(i_vmem, o_vmem):
            pltpu.sync_copy(table_hbm.at[i_vmem], o_vmem)

            @pl.loop(0, WINDOW)
            def _(r):
                for k in range(DIM // LANES):
                    sl = pl.ds(k * LANES, LANES)
                    o_vmem.at[r, sl][...] = o_vmem.at[r, sl][...] * SCALE

        pltpu.emit_pipeline(
            body,
            grid=(n // WINDOW,),
            in_specs=[pl.BlockSpec((WINDOW,), index_map=lambda i: (i,))],
            out_specs=[pl.BlockSpec((WINDOW, DIM), index_map=lambda i: (i, 0))],
            core_axis_name=("core", "subcore"),
            dimension_semantics=(pltpu.PARALLEL,),
        )(idx_hbm, out_hbm)

    out = emb_kernel(table, idx)
    return out.reshape(batch, seq, DIM)


# speedup vs baseline: 3.4140x; 1.5033x over previous
import functools
import math

import jax
import jax.numpy as jnp
from jax import lax
from jax.experimental import pallas as pl
from jax.experimental.pallas import tpu as pltpu
from jax.experimental.pallas import tpu_sc as plsc

DIM = 1024
SCALE = math.sqrt(DIM)
LANES = 16

NUM_WORKERS = 32
NBUF = 4
CHUNK = 16


def kernel(x, table):
    batch, seq = x.shape
    n = batch * seq
    rows_per_worker = n // NUM_WORKERS
    nchunks = rows_per_worker // CHUNK
    assert nchunks % NBUF == 0

    idx = x.reshape(n).astype(jnp.int32)
    mesh = plsc.VectorSubcoreMesh(
        core_axis_name="core", subcore_axis_name="subcore"
    )

    @functools.partial(
        pl.kernel,
        out_type=jax.ShapeDtypeStruct((n, DIM), jnp.float32),
        mesh=mesh,
        scratch_types=(
            [
                pltpu.VMEM((rows_per_worker,), jnp.int32),
                pltpu.VMEM((NBUF, CHUNK, DIM), jnp.float32),
            ]
            + [pltpu.SemaphoreType.DMA] * (2 * NBUF)
        ),
    )
    def emb_kernel(table_hbm, idx_hbm, out_hbm, idx_v, bufs, *sems):
        gsem = sems[:NBUF]
        ssem = sems[NBUF:]
        wid = lax.axis_index("subcore") * 2 + lax.axis_index("core")
        base = wid * rows_per_worker

        pltpu.sync_copy(idx_hbm.at[pl.ds(base, rows_per_worker)], idx_v)

        def gather_desc(j, b):
            return pltpu.make_async_copy(
                table_hbm.at[idx_v.at[pl.ds(j * CHUNK, CHUNK)]],
                bufs.at[b],
                gsem[b],
            )

        def scatter_desc(j, b):
            return pltpu.make_async_copy(
                bufs.at[b],
                out_hbm.at[pl.ds(base + j * CHUNK, CHUNK)],
                ssem[b],
            )

        gather_desc(0, 0).start()
        gather_desc(1, 1).start()

        @pl.loop(0, nchunks, step=NBUF)
        def _(g):
            for b in range(NBUF):
                j = g + b
                k = j + 2
                bk = (b + 2) % NBUF

                @pl.when(jnp.logical_and(k >= NBUF, k < nchunks))
                def _():
                    scatter_desc(k - NBUF, bk).wait()

                @pl.when(k < nchunks)
                def _():
                    gather_desc(k, bk).start()

                gather_desc(j, b).wait()

                @pl.loop(0, CHUNK)
                def _(r):
                    for v in range(DIM // LANES):
                        sl = pl.ds(v * LANES, LANES)
                        bufs.at[b, r, sl][...] = (
                            bufs.at[b, r, sl][...] * SCALE
                        )

                scatter_desc(j, b).start()

        for b in range(NBUF):
            scatter_desc(nchunks - NBUF + b, b).wait()

    out = emb_kernel(table, idx)
    return out.reshape(batch, seq, DIM)


# speedup vs baseline: 3.4907x; 1.0225x over previous
import functools
import math

import jax
import jax.numpy as jnp
from jax import lax
from jax.experimental import pallas as pl
from jax.experimental.pallas import tpu as pltpu
from jax.experimental.pallas import tpu_sc as plsc

DIM = 1024
SCALE = math.sqrt(DIM)
LANES = 16

NUM_WORKERS = 32
NBUF = 8
CHUNK = 8
AHEAD = 4


def kernel(x, table):
    batch, seq = x.shape
    n = batch * seq
    rows_per_worker = n // NUM_WORKERS
    nchunks = rows_per_worker // CHUNK
    assert nchunks % NBUF == 0

    idx = x.reshape(n).astype(jnp.int32)
    mesh = plsc.VectorSubcoreMesh(
        core_axis_name="core", subcore_axis_name="subcore"
    )

    @functools.partial(
        pl.kernel,
        out_type=jax.ShapeDtypeStruct((n, DIM), jnp.float32),
        mesh=mesh,
        scratch_types=(
            [
                pltpu.VMEM((rows_per_worker,), jnp.int32),
                pltpu.VMEM((NBUF, CHUNK, DIM), jnp.float32),
            ]
            + [pltpu.SemaphoreType.DMA] * (2 * NBUF)
        ),
    )
    def emb_kernel(table_hbm, idx_hbm, out_hbm, idx_v, bufs, *sems):
        gsem = sems[:NBUF]
        ssem = sems[NBUF:]
        wid = lax.axis_index("subcore") * 2 + lax.axis_index("core")
        base = wid * rows_per_worker

        pltpu.sync_copy(idx_hbm.at[pl.ds(base, rows_per_worker)], idx_v)

        def gather_desc(j, b):
            return pltpu.make_async_copy(
                table_hbm.at[idx_v.at[pl.ds(j * CHUNK, CHUNK)]],
                bufs.at[b],
                gsem[b],
            )

        def scatter_desc(j, b):
            return pltpu.make_async_copy(
                bufs.at[b],
                out_hbm.at[pl.ds(base + j * CHUNK, CHUNK)],
                ssem[b],
            )

        for j0 in range(AHEAD):
            gather_desc(j0, j0).start()

        @pl.loop(0, nchunks, step=NBUF)
        def _(g):
            for b in range(NBUF):
                j = g + b
                k = j + AHEAD
                bk = (b + AHEAD) % NBUF

                @pl.when(jnp.logical_and(k >= NBUF, k < nchunks))
                def _():
                    scatter_desc(k - NBUF, bk).wait()

                @pl.when(k < nchunks)
                def _():
                    gather_desc(k, bk).start()

                gather_desc(j, b).wait()

                @pl.loop(0, CHUNK)
                def _(r):
                    for v in range(DIM // LANES):
                        sl = pl.ds(v * LANES, LANES)
                        bufs.at[b, r, sl][...] = (
                            bufs.at[b, r, sl][...] * SCALE
                        )

                scatter_desc(j, b).start()

        for b in range(NBUF):
            scatter_desc(nchunks - NBUF + b, b).wait()

    out = emb_kernel(table, idx)
    return out.reshape(batch, seq, DIM)


# speedup vs baseline: 3.5252x; 1.0099x over previous
import functools
import math

import jax
import jax.numpy as jnp
from jax import lax
from jax.experimental import pallas as pl
from jax.experimental.pallas import tpu as pltpu
from jax.experimental.pallas import tpu_sc as plsc

DIM = 1024
SCALE = math.sqrt(DIM)
LANES = 16

NUM_WORKERS = 32
NBUF = 8
CHUNK = 8
AHEAD = 6


def kernel(x, table):
    batch, seq = x.shape
    n = batch * seq
    rows_per_worker = n // NUM_WORKERS
    nchunks = rows_per_worker // CHUNK
    assert nchunks % NBUF == 0

    idx = x.reshape(n).astype(jnp.int32)
    mesh = plsc.VectorSubcoreMesh(
        core_axis_name="core", subcore_axis_name="subcore"
    )

    @functools.partial(
        pl.kernel,
        out_type=jax.ShapeDtypeStruct((n, DIM), jnp.float32),
        mesh=mesh,
        scratch_types=(
            [
                pltpu.VMEM((rows_per_worker,), jnp.int32),
                pltpu.VMEM((NBUF, CHUNK, DIM), jnp.float32),
            ]
            + [pltpu.SemaphoreType.DMA] * (2 * NBUF)
        ),
    )
    def emb_kernel(table_hbm, idx_hbm, out_hbm, idx_v, bufs, *sems):
        gsem = sems[:NBUF]
        ssem = sems[NBUF:]
        wid = lax.axis_index("subcore") * 2 + lax.axis_index("core")
        base = wid * rows_per_worker

        pltpu.sync_copy(idx_hbm.at[pl.ds(base, rows_per_worker)], idx_v)

        def gather_desc(j, b):
            return pltpu.make_async_copy(
                table_hbm.at[idx_v.at[pl.ds(j * CHUNK, CHUNK)]],
                bufs.at[b],
                gsem[b],
            )

        def scatter_desc(j, b):
            return pltpu.make_async_copy(
                bufs.at[b],
                out_hbm.at[pl.ds(base + j * CHUNK, CHUNK)],
                ssem[b],
            )

        for j0 in range(AHEAD):
            gather_desc(j0, j0).start()

        @pl.loop(0, nchunks, step=NBUF)
        def _(g):
            for b in range(NBUF):
                j = g + b
                k = j + AHEAD
                bk = (b + AHEAD) % NBUF

                @pl.when(jnp.logical_and(k >= NBUF, k < nchunks))
                def _():
                    scatter_desc(k - NBUF, bk).wait()

                @pl.when(k < nchunks)
                def _():
                    gather_desc(k, bk).start()

                gather_desc(j, b).wait()

                @pl.loop(0, CHUNK)
                def _(r):
                    for v in range(DIM // LANES):
                        sl = pl.ds(v * LANES, LANES)
                        bufs.at[b, r, sl][...] = (
                            bufs.at[b, r, sl][...] * SCALE
                        )

                scatter_desc(j, b).start()

        for b in range(NBUF):
            scatter_desc(nchunks - NBUF + b, b).wait()

    out = emb_kernel(table, idx)
    return out.reshape(batch, seq, DIM)
